# Initial kernel scaffold; baseline (speedup 1.0000x reference)
import jax
import jax.numpy as jnp
from jax.experimental import pallas as pl


def kernel(x, idxs):
    raise NotImplementedError("write your pallas kernel here")


# speedup vs baseline: 1.2187x; 1.2187x over previous
import functools

import jax
import jax.numpy as jnp
from jax import lax
from jax.experimental import pallas as pl
from jax.experimental.pallas import tpu as pltpu
from jax.experimental.pallas import tpu_sc as plsc

_NC = 2
_NS = 16
_NW = _NC * _NS


def _sc_gather(x_flat, flat_idx, n_idx, d):
    per_w = n_idx // _NW
    chunk = 400
    n_chunks = per_w // chunk

    mesh = plsc.VectorSubcoreMesh(core_axis_name="c", subcore_axis_name="s")

    @functools.partial(
        pl.kernel,
        mesh=mesh,
        out_type=jax.ShapeDtypeStruct((n_idx, d), x_flat.dtype),
        scratch_types=[
            pltpu.VMEM((per_w,), jnp.int32),
            pltpu.VMEM((chunk, d), jnp.float32),
            pltpu.SemaphoreType.DMA,
        ],
    )
    def k(x_hbm, idx_hbm, out_hbm, idx_v, rows_v, sem):
        wid = lax.axis_index("s") * _NC + lax.axis_index("c")
        base = wid * per_w
        pltpu.sync_copy(idx_hbm.at[pl.ds(base, per_w)], idx_v)
        for c in range(n_chunks):
            pltpu.async_copy(
                x_hbm.at[idx_v.at[pl.ds(c * chunk, chunk)]], rows_v, sem
            ).wait()
            pltpu.sync_copy(rows_v, out_hbm.at[pl.ds(base + c * chunk, chunk)])

    return k(x_flat, flat_idx)


def kernel(x, idxs):
    b, n, d = x.shape
    k = idxs.shape[1]
    n_idx = b * k
    x_flat = x.reshape(b * n, d)
    flat_idx = (
        idxs.astype(jnp.int32) + (jnp.arange(b, dtype=jnp.int32) * n)[:, None]
    ).reshape(n_idx)
    out = _sc_gather(x_flat, flat_idx, n_idx, d)
    return out.reshape(b, k, d)


# speedup vs baseline: 1.2429x; 1.0198x over previous
import functools

import jax
import jax.numpy as jnp
from jax import lax
from jax.experimental import pallas as pl
from jax.experimental.pallas import tpu as pltpu
from jax.experimental.pallas import tpu_sc as plsc

_NC = 2
_NS = 16
_NW = _NC * _NS


def _sc_gather(x_flat, flat_idx, n_idx, d):
    per_w = n_idx // _NW
    chunk = 400
    n_chunks = per_w // chunk
    nbuf = 2

    mesh = plsc.VectorSubcoreMesh(core_axis_name="c", subcore_axis_name="s")

    @functools.partial(
        pl.kernel,
        mesh=mesh,
        out_type=jax.ShapeDtypeStruct((n_idx, d), x_flat.dtype),
        scratch_types=[
            pltpu.VMEM((per_w,), jnp.int32),
            pltpu.VMEM((nbuf, chunk, d), jnp.float32),
            pltpu.SemaphoreType.DMA,
            pltpu.SemaphoreType.DMA,
        ],
    )
    def k(x_hbm, idx_hbm, out_hbm, idx_v, rows_v, gsem, osem):
        wid = lax.axis_index("s") * _NC + lax.axis_index("c")
        base = wid * per_w
        pltpu.sync_copy(idx_hbm.at[pl.ds(base, per_w)], idx_v)

        def mk_g(c):
            return pltpu.make_async_copy(
                x_hbm.at[idx_v.at[pl.ds(c * chunk, chunk)]],
                rows_v.at[c % nbuf], gsem)

        def mk_o(c):
            return pltpu.make_async_copy(
                rows_v.at[c % nbuf],
                out_hbm.at[pl.ds(base + c * chunk, chunk)], osem)

        for c in range(min(nbuf, n_chunks)):
            mk_g(c).start()
        o_waited = 0
        for c in range(n_chunks):
            mk_g(c).wait()
            mk_o(c).start()
            if c + nbuf < n_chunks:
                mk_o(c).wait()
                o_waited += 1
                mk_g(c + nbuf).start()
        for c in range(n_chunks - o_waited):
            mk_o(c).wait()

    return k(x_flat, flat_idx)


def kernel(x, idxs):
    b, n, d = x.shape
    k = idxs.shape[1]
    n_idx = b * k
    x_flat = x.reshape(b * n, d)
    flat_idx = (
        idxs.astype(jnp.int32) + (jnp.arange(b, dtype=jnp.int32) * n)[:, None]
    ).reshape(n_idx)
    out = _sc_gather(x_flat, flat_idx, n_idx, d)
    return out.reshape(b, k, d)


# speedup vs baseline: 1.9065x; 1.5340x over previous
import functools

import jax
import jax.numpy as jnp
from jax import lax
from jax.experimental import pallas as pl
from jax.experimental.pallas import tpu as pltpu
from jax.experimental.pallas import tpu_sc as plsc

_NC = 2
_NS = 16
_NW = _NC * _NS


def _sc_gather(x_flat, idx_pad, b, k, d):
    per_w = b // _NW
    nbuf = 4
    kp = idx_pad.shape[1]

    mesh = plsc.VectorSubcoreMesh(core_axis_name="c", subcore_axis_name="s")

    @functools.partial(
        pl.kernel,
        mesh=mesh,
        out_type=jax.ShapeDtypeStruct((b, k, d), x_flat.dtype),
        scratch_types=[
            pltpu.VMEM((per_w, kp), jnp.int32),
            pltpu.VMEM((nbuf, k, d), jnp.float32),
            pltpu.SemaphoreType.DMA,
            pltpu.SemaphoreType.DMA,
        ],
    )
    def kern(x_hbm, idx_hbm, out_hbm, idx_v, rows_v, gsem, osem):
        wid = lax.axis_index("s") * _NC + lax.axis_index("c")
        bb = wid * per_w
        pltpu.sync_copy(idx_hbm.at[pl.ds(bb, per_w)], idx_v)

        def g_start(bi, j):
            pltpu.make_async_copy(
                x_hbm.at[idx_v.at[bi, pl.ds(0, k)]], rows_v.at[j], gsem
            ).start()

        def g_wait():
            pltpu.make_async_copy(
                x_hbm.at[idx_v.at[0, pl.ds(0, k)]], rows_v.at[0], gsem
            ).wait()

        def o_start(bi, j):
            pltpu.make_async_copy(rows_v.at[j], out_hbm.at[bb + bi], osem).start()

        def o_wait():
            pltpu.make_async_copy(rows_v.at[0], out_hbm.at[bb], osem).wait()

        for j in range(nbuf):
            g_start(j, j)

        @pl.loop(0, per_w // nbuf - 1)
        def _(c):
            for j in range(nbuf):
                bi = c * nbuf + j
                g_wait()
                o_start(bi, j)
                o_wait()
                g_start(bi + nbuf, j)

        for j in range(nbuf):
            bi = per_w - nbuf + j
            g_wait()
            o_start(bi, j)
        for j in range(nbuf):
            o_wait()

    return kern(x_flat, idx_pad)


def kernel(x, idxs):
    b, n, d = x.shape
    k = idxs.shape[1]
    x_flat = x.reshape(b * n, d)
    fidx = idxs.astype(jnp.int32) + (jnp.arange(b, dtype=jnp.int32) * n)[:, None]
    kp = (k + 7) // 8 * 8
    idx_pad = jnp.pad(fidx, ((0, 0), (0, kp - k)), mode="edge")
    return _sc_gather(x_flat, idx_pad, b, k, d)


# speedup vs baseline: 3.0341x; 1.5914x over previous
import functools

import jax
import jax.numpy as jnp
from jax import lax
from jax.experimental import pallas as pl
from jax.experimental.pallas import tpu as pltpu
from jax.experimental.pallas import tpu_sc as plsc

_NC = 2
_NS = 16
_NW = _NC * _NS


def _sc_gather(x_flat, flat_idx, n_idx, d):
    per_w = n_idx // _NW
    chunk = 200
    n_chunks = per_w // chunk
    nbuf = 4

    mesh = plsc.VectorSubcoreMesh(core_axis_name="c", subcore_axis_name="s")

    @functools.partial(
        pl.kernel,
        mesh=mesh,
        out_type=jax.ShapeDtypeStruct((n_idx, d), x_flat.dtype),
        scratch_types=[
            pltpu.VMEM((per_w,), jnp.int32),
            pltpu.VMEM((nbuf, chunk, d), jnp.float32),
            pltpu.SemaphoreType.DMA,
            pltpu.SemaphoreType.DMA,
        ],
    )
    def kern(x_hbm, idx_hbm, out_hbm, idx_v, rows_v, gsem, osem):
        wid = lax.axis_index("s") * _NC + lax.axis_index("c")
        base = wid * per_w
        pltpu.sync_copy(idx_hbm.at[pl.ds(base, per_w)], idx_v)

        def mk_g(c):
            return pltpu.make_async_copy(
                x_hbm.at[idx_v.at[pl.ds(c * chunk, chunk)]],
                rows_v.at[c % nbuf], gsem)

        def mk_o(c):
            return pltpu.make_async_copy(
                rows_v.at[c % nbuf],
                out_hbm.at[pl.ds(base + c * chunk, chunk)], osem)

        for c in range(min(nbuf, n_chunks)):
            mk_g(c).start()
        o_waited = 0
        for c in range(n_chunks):
            mk_g(c).wait()
            mk_o(c).start()
            if c + nbuf < n_chunks:
                mk_o(c).wait()
                o_waited += 1
                mk_g(c + nbuf).start()
        for c in range(n_chunks - o_waited):
            mk_o(c).wait()

    return kern(x_flat, flat_idx)


def kernel(x, idxs):
    b, n, d = x.shape
    k = idxs.shape[1]
    n_idx = b * k
    x_flat = x.reshape(b * n, d)
    tidx = (
        idxs.astype(jnp.int32) + (jnp.arange(b, dtype=jnp.int32) * n)[:, None]
    ).T.reshape(n_idx)
    out = _sc_gather(x_flat, tidx, n_idx, d)
    return out.reshape(k, b, d).transpose(1, 0, 2)
